# Initial kernel scaffold; baseline (speedup 1.0000x reference)
import jax
import jax.numpy as jnp
from jax.experimental import pallas as pl


def kernel(h, e, u, edge_index, node2graph, WA, bA, WB, bB, WC, bC, WD, bD, WE, bE, WF, bF, WG, bG, WH, bH, WI, bI, gamma_h, beta_h, gamma_e, beta_e, gamma_u, beta_u):
    raise NotImplementedError("write your pallas kernel here")


# speedup vs baseline: 2.0159x; 2.0159x over previous
import functools

import jax
import jax.numpy as jnp
from jax import lax
from jax.experimental import pallas as pl
from jax.experimental.pallas import tpu as pltpu
from jax.experimental.pallas import tpu_sc as plsc

N = 10000
E = 320000
B = 128
D = 128

NC = 2
NS = 16
NW = NC * NS
EW = E // NW
ES = E // NS
KE = 80
LANES = 16

F32 = jnp.float32


def _tables_body(h_ref, u_ref, n2g_ref, WA_ref, bA_ref, WC_ref, bC_ref,
                 WE_ref, bE_ref, WD_ref, bD_ref, WF_ref, bF_ref,
                 t1_ref, t2_ref, eh_ref, xb_ref):
    h = h_ref[...]
    u = u_ref[...]
    ah = jnp.dot(h, WA_ref[...], preferred_element_type=F32) + bA_ref[...]
    cu = jnp.dot(u, WC_ref[...], preferred_element_type=F32) + bC_ref[...]
    fu = jnp.dot(u, WF_ref[...], preferred_element_type=F32) + bF_ref[...]
    oh = (n2g_ref[...] == lax.broadcasted_iota(jnp.int32, (N, B), 1)).astype(F32)
    cua = jnp.dot(oh, cu, preferred_element_type=F32)
    h2 = jnp.dot(oh, fu, preferred_element_type=F32)
    dh = jnp.dot(h, WD_ref[...], preferred_element_type=F32) + bD_ref[...]
    t2_ref[...] = ah
    t1_ref[...] = ah + cua
    eh_ref[...] = jnp.dot(h, WE_ref[...], preferred_element_type=F32) + bE_ref[...]
    xb_ref[...] = dh + h2


def _tables_call(h, u, n2g_col, WA, bA, WC, bC, WE, bE, WD, bD, WF, bF):
    out = jax.ShapeDtypeStruct((N, D), F32)
    return pl.pallas_call(
        _tables_body,
        out_shape=(out, out, out, out),
    )(h, u, n2g_col, WA, bA, WC, bC, WE, bE, WD, bD, WF, bF)


_CE = 2000


def _z_body(e_ref, WB_ref, bB_ref, z_ref):
    z_ref[...] = jnp.dot(e_ref[...], WB_ref[...], preferred_element_type=F32) + bB_ref[...]


def _z_call(e, WB, bB):
    return pl.pallas_call(
        _z_body,
        grid=(E // _CE,),
        in_specs=[
            pl.BlockSpec((_CE, D), lambda i: (i, 0)),
            pl.BlockSpec((D, D), lambda i: (0, 0)),
            pl.BlockSpec((1, D), lambda i: (0, 0)),
        ],
        out_specs=pl.BlockSpec((_CE, D), lambda i: (i, 0)),
        out_shape=jax.ShapeDtypeStruct((E, D), F32),
    )(e, WB, bB)


def _sc_gather_body(z_hbm, src_hbm, dst_hbm, t1_hbm, t2_hbm,
                    r_hbm, stats_hbm,
                    src_v, dst_v, g1_v, g2_v, zb_v, acc_v, sem1, sem2):
    c = lax.axis_index("c")
    s = lax.axis_index("s")
    wid = s * NC + c
    base_w = wid * EW
    zero = jnp.zeros((LANES,), F32)
    for cc in range(D // LANES):
        acc_v[0, pl.ds(cc * LANES, LANES)] = zero
        acc_v[1, pl.ds(cc * LANES, LANES)] = zero

    def chunk(j, carry):
        base = base_w + j * KE
        pltpu.sync_copy(src_hbm.at[pl.ds(base, KE)], src_v)
        pltpu.sync_copy(dst_hbm.at[pl.ds(base, KE)], dst_v)
        cp1 = pltpu.async_copy(t1_hbm.at[src_v], g1_v, sem1)
        cp2 = pltpu.async_copy(t2_hbm.at[dst_v], g2_v, sem2)
        pltpu.sync_copy(z_hbm.at[pl.ds(base, KE)], zb_v)
        cp1.wait()
        cp2.wait()

        def row(i, rc):
            for cc in range(D // LANES):
                sl = pl.ds(cc * LANES, LANES)
                r = zb_v[i, sl] + g1_v[i, sl] + g2_v[i, sl]
                zb_v[i, sl] = r
                acc_v[0, sl] = acc_v[0, sl] + r
                acc_v[1, sl] = acc_v[1, sl] + r * r
            return rc

        lax.fori_loop(0, KE, row, 0, unroll=False)
        pltpu.sync_copy(zb_v, r_hbm.at[pl.ds(base, KE)])
        return carry

    lax.fori_loop(0, EW // KE, chunk, 0, unroll=False)
    pltpu.sync_copy(acc_v, stats_hbm.at[wid])


def _sc_gather_call(z, src, dst, t1, t2):
    mesh = plsc.VectorSubcoreMesh(core_axis_name="c", subcore_axis_name="s")
    fn = functools.partial(
        pl.kernel,
        out_type=(
            jax.ShapeDtypeStruct((E, D), F32),
            jax.ShapeDtypeStruct((NW, 2, D), F32),
        ),
        mesh=mesh,
        scratch_types=(
            pltpu.VMEM((KE,), jnp.int32),
            pltpu.VMEM((KE,), jnp.int32),
            pltpu.VMEM((KE, D), F32),
            pltpu.VMEM((KE, D), F32),
            pltpu.VMEM((KE, D), F32),
            pltpu.VMEM((2, D), F32),
            pltpu.SemaphoreType.DMA,
            pltpu.SemaphoreType.DMA,
        ),
    )(_sc_gather_body)
    return fn(z, src, dst, t1, t2)


def _enew_body(r_ref, stats_ref, ge_ref, be_ref, en_ref, sg_ref):
    st = stats_ref[...]
    s0 = jnp.sum(st[:, 0, :], axis=0)
    s1 = jnp.sum(st[:, 1, :], axis=0)
    mu = s0 / float(E)
    var = s1 / float(E) - mu * mu
    rstd = lax.rsqrt(var + 1e-5)
    scale = (ge_ref[...][0] * rstd)[None, :]
    shift = (be_ref[...][0] - mu * ge_ref[...][0] * rstd)[None, :]
    en = jnp.maximum(r_ref[...] * scale + shift, 0.0)
    en_ref[...] = en
    sg_ref[...] = 1.0 / (1.0 + jnp.exp(-en))


def _enew_call(r, stats, gamma_e, beta_e):
    return pl.pallas_call(
        _enew_body,
        grid=(E // _CE,),
        in_specs=[
            pl.BlockSpec((_CE, D), lambda i: (i, 0)),
            pl.BlockSpec((NW, 2, D), lambda i: (0, 0, 0)),
            pl.BlockSpec((1, D), lambda i: (0, 0)),
            pl.BlockSpec((1, D), lambda i: (0, 0)),
        ],
        out_specs=(
            pl.BlockSpec((_CE, D), lambda i: (i, 0)),
            pl.BlockSpec((_CE, D), lambda i: (i, 0)),
        ),
        out_shape=(
            jax.ShapeDtypeStruct((E, D), F32),
            jax.ShapeDtypeStruct((E, D), F32),
        ),
    )(r, stats, gamma_e, beta_e)


_NZC = (N // KE + NS - 1) // NS


def _sc_scatter_body(sig_hbm, en_hbm, src_hbm, dst_hbm, n2g_hbm, eh_hbm,
                     num_hbm, den_hbm, sb_hbm, mc_hbm,
                     tbl, sbt, mct,
                     src_v, dst_v, g2_v, a_v, b_v, ones_v, z16_v,
                     sem1, sem2):
    c = lax.axis_index("c")
    s = lax.axis_index("s")
    sid = s
    zero = jnp.zeros((LANES,), F32)
    one = jnp.full((LANES,), 1.0, F32)

    def zrow(i, carry):
        for cc in range(D // LANES):
            a_v[i, pl.ds(cc * LANES, LANES)] = zero
        return carry

    lax.fori_loop(0, KE, zrow, 0, unroll=False)

    def orow(i, carry):
        ones_v[i, :] = one
        return carry

    lax.fori_loop(0, KE, orow, 0, unroll=False)

    def z16row(i, carry):
        z16_v[i, :] = zero
        return carry

    lax.fori_loop(0, B, z16row, 0, unroll=False)

    nzc = N // KE

    def zchunk(k, carry):
        j = sid + k * NS

        @pl.when(j < nzc)
        def _():
            pltpu.sync_copy(a_v, tbl.at[pl.ds(j * KE, KE)])

        return carry

    lax.fori_loop(0, _NZC, zchunk, 0, unroll=False)

    @pl.when(sid == 0)
    def _():
        pltpu.sync_copy(a_v, sbt.at[pl.ds(0, KE)])
        pltpu.sync_copy(a_v, sbt.at[pl.ds(B - KE, KE)])
        pltpu.sync_copy(z16_v, mct)

    plsc.subcore_barrier()

    nchunk = ES // KE

    @pl.when(c == 0)
    def _():
        def chunk(j, carry):
            base = s * ES + j * KE
            pltpu.sync_copy(src_hbm.at[pl.ds(base, KE)], src_v)
            pltpu.sync_copy(dst_hbm.at[pl.ds(base, KE)], dst_v)
            cp = pltpu.async_copy(eh_hbm.at[src_v], b_v, sem1)
            pltpu.sync_copy(sig_hbm.at[pl.ds(base, KE)], a_v)
            cp.wait()

            def row(i, rc):
                for cc in range(D // LANES):
                    sl = pl.ds(cc * LANES, LANES)
                    a_v[i, sl] = a_v[i, sl] * b_v[i, sl]
                return rc

            lax.fori_loop(0, KE, row, 0, unroll=False)
            pltpu.sync_copy(a_v, tbl.at[dst_v], add=True)
            return carry

        lax.fori_loop(0, nchunk, chunk, 0, unroll=False)

    @pl.when(c == 1)
    def _():
        def chunk(j, carry):
            base = s * ES + j * KE
            pltpu.sync_copy(dst_hbm.at[pl.ds(base, KE)], dst_v)
            cpg = pltpu.async_copy(n2g_hbm.at[dst_v], g2_v, sem2)
            pltpu.sync_copy(sig_hbm.at[pl.ds(base, KE)], a_v)
            pltpu.sync_copy(en_hbm.at[pl.ds(base, KE)], b_v)
            pltpu.sync_copy(a_v, tbl.at[dst_v], add=True)
            cpg.wait()
            pltpu.sync_copy(b_v, sbt.at[g2_v], add=True)
            pltpu.sync_copy(ones_v, mct.at[g2_v], add=True)
            return carry

        lax.fori_loop(0, nchunk, chunk, 0, unroll=False)

    plsc.subcore_barrier()

    @pl.when(sid == 0)
    def _():
        @pl.when(c == 0)
        def _():
            pltpu.sync_copy(tbl, num_hbm)

        @pl.when(c == 1)
        def _():
            pltpu.sync_copy(tbl, den_hbm)
            pltpu.sync_copy(sbt, sb_hbm)
            pltpu.sync_copy(mct, mc_hbm)


def _sc_scatter_call(sig, en, src, dst, n2g, eh):
    mesh = plsc.VectorSubcoreMesh(core_axis_name="c", subcore_axis_name="s")
    fn = functools.partial(
        pl.kernel,
        out_type=(
            jax.ShapeDtypeStruct((N, D), F32),
            jax.ShapeDtypeStruct((N, D), F32),
            jax.ShapeDtypeStruct((B, D), F32),
            jax.ShapeDtypeStruct((B, LANES), F32),
        ),
        mesh=mesh,
        scratch_types=(
            pltpu.VMEM_SHARED((N, D), F32),
            pltpu.VMEM_SHARED((B, D), F32),
            pltpu.VMEM_SHARED((B, LANES), F32),
            pltpu.VMEM((KE,), jnp.int32),
            pltpu.VMEM((KE,), jnp.int32),
            pltpu.VMEM((KE,), jnp.int32),
            pltpu.VMEM((KE, D), F32),
            pltpu.VMEM((KE, D), F32),
            pltpu.VMEM((KE, LANES), F32),
            pltpu.VMEM((B, LANES), F32),
            pltpu.SemaphoreType.DMA,
            pltpu.SemaphoreType.DMA,
        ),
    )(_sc_scatter_body)
    return fn(sig, en, src, dst, n2g, eh)


def _final_body(num_ref, den_ref, sb_ref, mc_ref, xb_ref, n2g_ref, u_ref,
                WG_ref, bG_ref, WH_ref, bH_ref, WI_ref, bI_ref,
                gh_ref, bh_ref, gu_ref, bu_ref,
                hn_ref, un_ref):
    x = xb_ref[...] + num_ref[...] / (den_ref[...] + 1e-6)
    mu = jnp.mean(x, axis=0, keepdims=True)
    var = jnp.mean(x * x, axis=0, keepdims=True) - mu * mu
    hn = gh_ref[...] * (x - mu) * lax.rsqrt(var + 1e-5) + bh_ref[...]
    hn = jnp.maximum(hn, 0.0)
    hn_ref[...] = hn

    ohT = (lax.broadcasted_iota(jnp.int32, (B, N), 0) == n2g_ref[...]).astype(F32)
    M = jnp.dot(ohT, hn, preferred_element_type=F32)
    cnt = jnp.sum(ohT, axis=1, keepdims=True)
    Gg = jnp.dot(M, WG_ref[...], preferred_element_type=F32) + cnt * bG_ref[...]
    mean_Gh = Gg / jnp.maximum(cnt, 1.0)
    mg = jnp.sum(mc_ref[...], axis=1, keepdims=True) * (1.0 / LANES)
    He_g = jnp.dot(sb_ref[...], WH_ref[...], preferred_element_type=F32) + mg * bH_ref[...]
    mean_He = He_g * (1.0 / float(E))
    un = mean_Gh + mean_He + jnp.dot(u_ref[...], WI_ref[...], preferred_element_type=F32) + bI_ref[...]
    mu_u = jnp.mean(un, axis=0, keepdims=True)
    var_u = jnp.mean(un * un, axis=0, keepdims=True) - mu_u * mu_u
    un = gu_ref[...] * (un - mu_u) * lax.rsqrt(var_u + 1e-5) + bu_ref[...]
    un_ref[...] = jnp.maximum(un, 0.0)


def _final_call(num, den, sb, mc, xb, n2g_row, u,
                WG, bG, WH, bH, WI, bI, gamma_h, beta_h, gamma_u, beta_u):
    return pl.pallas_call(
        _final_body,
        out_shape=(
            jax.ShapeDtypeStruct((N, D), F32),
            jax.ShapeDtypeStruct((B, D), F32),
        ),
    )(num, den, sb, mc, xb, n2g_row, u,
      WG, bG, WH, bH, WI, bI, gamma_h, beta_h, gamma_u, beta_u)


def kernel(h, e, u, edge_index, node2graph,
           WA, bA, WB, bB, WC, bC, WD, bD, WE, bE, WF, bF, WG, bG, WH, bH,
           WI, bI, gamma_h, beta_h, gamma_e, beta_e, gamma_u, beta_u):
    src = edge_index[0]
    dst = edge_index[1]
    n2g_col = node2graph[:, None]
    n2g_row = node2graph[None, :]
    r2 = lambda v: v[None, :]

    t1, t2, eh, xb = _tables_call(
        h, u, n2g_col, WA, r2(bA), WC, r2(bC), WE, r2(bE), WD, r2(bD),
        WF, r2(bF))
    z = _z_call(e, WB, r2(bB))
    r, stats = _sc_gather_call(z, src, dst, t1, t2)
    e_new, sig = _enew_call(r, stats, r2(gamma_e), r2(beta_e))
    num, den, sb, mc = _sc_scatter_call(sig, e_new, src, dst, node2graph, eh)
    h_new, u_new = _final_call(
        num, den, sb, mc, xb, n2g_row, u,
        WG, r2(bG), WH, r2(bH), WI, r2(bI),
        r2(gamma_h), r2(beta_h), r2(gamma_u), r2(beta_u))
    return (h_new, e_new, u_new)


# speedup vs baseline: 2.3052x; 1.1435x over previous
import functools

import jax
import jax.numpy as jnp
from jax import lax
from jax.experimental import pallas as pl
from jax.experimental.pallas import tpu as pltpu
from jax.experimental.pallas import tpu_sc as plsc

N = 10000
E = 320000
B = 128
D = 128

NC = 2
NS = 16
NW = NC * NS
EW = E // NW
ES = E // NS
KG = 200
KE = 80
LANES = 16

F32 = jnp.float32


def _tables_body(h_ref, u_ref, n2g_ref, WA_ref, bA_ref, WC_ref, bC_ref,
                 WE_ref, bE_ref, WD_ref, bD_ref, WF_ref, bF_ref,
                 t1_ref, t2_ref, eh_ref, xb_ref):
    h = h_ref[...]
    u = u_ref[...]
    ah = jnp.dot(h, WA_ref[...], preferred_element_type=F32) + bA_ref[...]
    cu = jnp.dot(u, WC_ref[...], preferred_element_type=F32) + bC_ref[...]
    fu = jnp.dot(u, WF_ref[...], preferred_element_type=F32) + bF_ref[...]
    oh = (n2g_ref[...] == lax.broadcasted_iota(jnp.int32, (N, B), 1)).astype(F32)
    cua = jnp.dot(oh, cu, preferred_element_type=F32)
    h2 = jnp.dot(oh, fu, preferred_element_type=F32)
    dh = jnp.dot(h, WD_ref[...], preferred_element_type=F32) + bD_ref[...]
    t2_ref[...] = ah
    t1_ref[...] = ah + cua
    eh_ref[...] = jnp.dot(h, WE_ref[...], preferred_element_type=F32) + bE_ref[...]
    xb_ref[...] = dh + h2


def _tables_call(h, u, n2g_col, WA, bA, WC, bC, WE, bE, WD, bD, WF, bF):
    out = jax.ShapeDtypeStruct((N, D), F32)
    return pl.pallas_call(
        _tables_body,
        out_shape=(out, out, out, out),
    )(h, u, n2g_col, WA, bA, WC, bC, WE, bE, WD, bD, WF, bF)


_CE = 2000


def _z_body(e_ref, WB_ref, bB_ref, z_ref):
    z_ref[...] = jnp.dot(e_ref[...], WB_ref[...], preferred_element_type=F32) + bB_ref[...]


def _z_call(e, WB, bB):
    return pl.pallas_call(
        _z_body,
        grid=(E // _CE,),
        in_specs=[
            pl.BlockSpec((_CE, D), lambda i: (i, 0)),
            pl.BlockSpec((D, D), lambda i: (0, 0)),
            pl.BlockSpec((1, D), lambda i: (0, 0)),
        ],
        out_specs=pl.BlockSpec((_CE, D), lambda i: (i, 0)),
        out_shape=jax.ShapeDtypeStruct((E, D), F32),
    )(e, WB, bB)


def _sc_gather_body(z_hbm, src_hbm, dst_hbm, t1_hbm, t2_hbm,
                    r_hbm, stats_hbm,
                    src_v, dst_v, g1_v, g2_v, zb_v, acc_v, sem1, sem2, sem3):
    c = lax.axis_index("c")
    s = lax.axis_index("s")
    wid = s * NC + c
    base_w = wid * EW
    zero = jnp.zeros((LANES,), F32)
    for cc in range(D // LANES):
        acc_v[0, pl.ds(cc * LANES, LANES)] = zero
        acc_v[1, pl.ds(cc * LANES, LANES)] = zero

    def chunk(j, carry):
        base = base_w + j * KG
        ci1 = pltpu.async_copy(src_hbm.at[pl.ds(base, KG)], src_v, sem1)
        ci2 = pltpu.async_copy(dst_hbm.at[pl.ds(base, KG)], dst_v, sem2)
        cpz = pltpu.async_copy(z_hbm.at[pl.ds(base, KG)], zb_v, sem3)
        ci1.wait()
        ci2.wait()
        cp1 = pltpu.async_copy(t1_hbm.at[src_v], g1_v, sem1)
        cp2 = pltpu.async_copy(t2_hbm.at[dst_v], g2_v, sem2)
        cpz.wait()
        cp1.wait()
        cp2.wait()

        def row(i, rc):
            for cc in range(D // LANES):
                sl = pl.ds(cc * LANES, LANES)
                r = zb_v[i, sl] + g1_v[i, sl] + g2_v[i, sl]
                zb_v[i, sl] = r
                acc_v[0, sl] = acc_v[0, sl] + r
                acc_v[1, sl] = acc_v[1, sl] + r * r
            return rc

        lax.fori_loop(0, KG, row, 0, unroll=False)
        pltpu.sync_copy(zb_v, r_hbm.at[pl.ds(base, KG)])
        return carry

    lax.fori_loop(0, EW // KG, chunk, 0, unroll=False)
    pltpu.sync_copy(acc_v, stats_hbm.at[wid])


def _sc_gather_call(z, src, dst, t1, t2):
    mesh = plsc.VectorSubcoreMesh(core_axis_name="c", subcore_axis_name="s")
    fn = functools.partial(
        pl.kernel,
        out_type=(
            jax.ShapeDtypeStruct((E, D), F32),
            jax.ShapeDtypeStruct((NW, 2, D), F32),
        ),
        mesh=mesh,
        scratch_types=(
            pltpu.VMEM((KG,), jnp.int32),
            pltpu.VMEM((KG,), jnp.int32),
            pltpu.VMEM((KG, D), F32),
            pltpu.VMEM((KG, D), F32),
            pltpu.VMEM((KG, D), F32),
            pltpu.VMEM((2, D), F32),
            pltpu.SemaphoreType.DMA,
            pltpu.SemaphoreType.DMA,
            pltpu.SemaphoreType.DMA,
        ),
    )(_sc_gather_body)
    return fn(z, src, dst, t1, t2)


def _enew_body(r_ref, stats_ref, ge_ref, be_ref, en_ref, sg_ref):
    st = stats_ref[...]
    s0 = jnp.sum(st[:, 0, :], axis=0)
    s1 = jnp.sum(st[:, 1, :], axis=0)
    mu = s0 / float(E)
    var = s1 / float(E) - mu * mu
    rstd = lax.rsqrt(var + 1e-5)
    scale = (ge_ref[...][0] * rstd)[None, :]
    shift = (be_ref[...][0] - mu * ge_ref[...][0] * rstd)[None, :]
    en = jnp.maximum(r_ref[...] * scale + shift, 0.0)
    en_ref[...] = en
    sg_ref[...] = 1.0 / (1.0 + jnp.exp(-en))


def _enew_call(r, stats, gamma_e, beta_e):
    return pl.pallas_call(
        _enew_body,
        grid=(E // _CE,),
        in_specs=[
            pl.BlockSpec((_CE, D), lambda i: (i, 0)),
            pl.BlockSpec((NW, 2, D), lambda i: (0, 0, 0)),
            pl.BlockSpec((1, D), lambda i: (0, 0)),
            pl.BlockSpec((1, D), lambda i: (0, 0)),
        ],
        out_specs=(
            pl.BlockSpec((_CE, D), lambda i: (i, 0)),
            pl.BlockSpec((_CE, D), lambda i: (i, 0)),
        ),
        out_shape=(
            jax.ShapeDtypeStruct((E, D), F32),
            jax.ShapeDtypeStruct((E, D), F32),
        ),
    )(r, stats, gamma_e, beta_e)


_ZC = 80
_NZC = (N // _ZC + NS - 1) // NS


def _sc_scatter_body(sig_hbm, en_hbm, src_hbm, dst_hbm, n2g_hbm, eh_hbm,
                     num_hbm, den_hbm, sb_hbm, mc_hbm,
                     tbl, sbt, mct,
                     src_v, dst_v, g2_v, a_v, b_v, ones_v, z16_v,
                     sem1, sem2, sem3):
    c = lax.axis_index("c")
    s = lax.axis_index("s")
    sid = s
    zero = jnp.zeros((LANES,), F32)
    one = jnp.full((LANES,), 1.0, F32)

    def zrow(i, carry):
        for cc in range(D // LANES):
            a_v[i, pl.ds(cc * LANES, LANES)] = zero
        return carry

    lax.fori_loop(0, KE, zrow, 0, unroll=False)

    def orow(i, carry):
        ones_v[i, :] = one
        return carry

    lax.fori_loop(0, KE, orow, 0, unroll=False)

    def z16row(i, carry):
        z16_v[i, :] = zero
        return carry

    lax.fori_loop(0, B, z16row, 0, unroll=False)

    nzc = N // _ZC

    def zchunk(k, carry):
        j = sid + k * NS

        @pl.when(j < nzc)
        def _():
            pltpu.sync_copy(a_v.at[pl.ds(0, _ZC)], tbl.at[pl.ds(j * _ZC, _ZC)])

        return carry

    lax.fori_loop(0, _NZC, zchunk, 0, unroll=False)

    @pl.when(sid == 0)
    def _():
        pltpu.sync_copy(a_v.at[pl.ds(0, B)], sbt)
        pltpu.sync_copy(z16_v, mct)

    plsc.subcore_barrier()

    nchunk = ES // KE

    @pl.when(c == 0)
    def _():
        def chunk(j, carry):
            base = s * ES + j * KE
            ci1 = pltpu.async_copy(src_hbm.at[pl.ds(base, KE)], src_v, sem1)
            ci2 = pltpu.async_copy(dst_hbm.at[pl.ds(base, KE)], dst_v, sem2)
            cpa = pltpu.async_copy(sig_hbm.at[pl.ds(base, KE)], a_v, sem3)
            ci1.wait()
            cp = pltpu.async_copy(eh_hbm.at[src_v], b_v, sem1)
            ci2.wait()
            cpa.wait()
            cp.wait()

            def row(i, rc):
                for cc in range(D // LANES):
                    sl = pl.ds(cc * LANES, LANES)
                    a_v[i, sl] = a_v[i, sl] * b_v[i, sl]
                return rc

            lax.fori_loop(0, KE, row, 0, unroll=False)
            pltpu.sync_copy(a_v, tbl.at[dst_v], add=True)
            return carry

        lax.fori_loop(0, nchunk, chunk, 0, unroll=False)

    @pl.when(c == 1)
    def _():
        def chunk(j, carry):
            base = s * ES + j * KE
            ci2 = pltpu.async_copy(dst_hbm.at[pl.ds(base, KE)], dst_v, sem2)
            cpa = pltpu.async_copy(sig_hbm.at[pl.ds(base, KE)], a_v, sem3)
            cpb = pltpu.async_copy(en_hbm.at[pl.ds(base, KE)], b_v, sem1)
            ci2.wait()
            cpg = pltpu.async_copy(n2g_hbm.at[dst_v], g2_v, sem2)
            cpa.wait()
            pltpu.sync_copy(a_v, tbl.at[dst_v], add=True)
            cpg.wait()
            cpb.wait()
            pltpu.sync_copy(b_v, sbt.at[g2_v], add=True)
            pltpu.sync_copy(ones_v, mct.at[g2_v], add=True)
            return carry

        lax.fori_loop(0, nchunk, chunk, 0, unroll=False)

    plsc.subcore_barrier()

    @pl.when(sid == 0)
    def _():
        @pl.when(c == 0)
        def _():
            pltpu.sync_copy(tbl, num_hbm)

        @pl.when(c == 1)
        def _():
            pltpu.sync_copy(tbl, den_hbm)
            pltpu.sync_copy(sbt, sb_hbm)
            pltpu.sync_copy(mct, mc_hbm)


def _sc_scatter_call(sig, en, src, dst, n2g, eh):
    mesh = plsc.VectorSubcoreMesh(core_axis_name="c", subcore_axis_name="s")
    fn = functools.partial(
        pl.kernel,
        out_type=(
            jax.ShapeDtypeStruct((N, D), F32),
            jax.ShapeDtypeStruct((N, D), F32),
            jax.ShapeDtypeStruct((B, D), F32),
            jax.ShapeDtypeStruct((B, LANES), F32),
        ),
        mesh=mesh,
        scratch_types=(
            pltpu.VMEM_SHARED((N, D), F32),
            pltpu.VMEM_SHARED((B, D), F32),
            pltpu.VMEM_SHARED((B, LANES), F32),
            pltpu.VMEM((KE,), jnp.int32),
            pltpu.VMEM((KE,), jnp.int32),
            pltpu.VMEM((KE,), jnp.int32),
            pltpu.VMEM((KE, D), F32),
            pltpu.VMEM((KE, D), F32),
            pltpu.VMEM((KE, LANES), F32),
            pltpu.VMEM((B, LANES), F32),
            pltpu.SemaphoreType.DMA,
            pltpu.SemaphoreType.DMA,
            pltpu.SemaphoreType.DMA,
        ),
    )(_sc_scatter_body)
    return fn(sig, en, src, dst, n2g, eh)


def _final_body(num_ref, den_ref, sb_ref, mc_ref, xb_ref, n2g_ref, u_ref,
                WG_ref, bG_ref, WH_ref, bH_ref, WI_ref, bI_ref,
                gh_ref, bh_ref, gu_ref, bu_ref,
                hn_ref, un_ref):
    x = xb_ref[...] + num_ref[...] / (den_ref[...] + 1e-6)
    mu = jnp.mean(x, axis=0, keepdims=True)
    var = jnp.mean(x * x, axis=0, keepdims=True) - mu * mu
    hn = gh_ref[...] * (x - mu) * lax.rsqrt(var + 1e-5) + bh_ref[...]
    hn = jnp.maximum(hn, 0.0)
    hn_ref[...] = hn

    ohT = (lax.broadcasted_iota(jnp.int32, (B, N), 0) == n2g_ref[...]).astype(F32)
    M = jnp.dot(ohT, hn, preferred_element_type=F32)
    cnt = jnp.sum(ohT, axis=1, keepdims=True)
    Gg = jnp.dot(M, WG_ref[...], preferred_element_type=F32) + cnt * bG_ref[...]
    mean_Gh = Gg / jnp.maximum(cnt, 1.0)
    mg = jnp.sum(mc_ref[...], axis=1, keepdims=True) * (1.0 / LANES)
    He_g = jnp.dot(sb_ref[...], WH_ref[...], preferred_element_type=F32) + mg * bH_ref[...]
    mean_He = He_g * (1.0 / float(E))
    un = mean_Gh + mean_He + jnp.dot(u_ref[...], WI_ref[...], preferred_element_type=F32) + bI_ref[...]
    mu_u = jnp.mean(un, axis=0, keepdims=True)
    var_u = jnp.mean(un * un, axis=0, keepdims=True) - mu_u * mu_u
    un = gu_ref[...] * (un - mu_u) * lax.rsqrt(var_u + 1e-5) + bu_ref[...]
    un_ref[...] = jnp.maximum(un, 0.0)


def _final_call(num, den, sb, mc, xb, n2g_row, u,
                WG, bG, WH, bH, WI, bI, gamma_h, beta_h, gamma_u, beta_u):
    return pl.pallas_call(
        _final_body,
        out_shape=(
            jax.ShapeDtypeStruct((N, D), F32),
            jax.ShapeDtypeStruct((B, D), F32),
        ),
    )(num, den, sb, mc, xb, n2g_row, u,
      WG, bG, WH, bH, WI, bI, gamma_h, beta_h, gamma_u, beta_u)


def kernel(h, e, u, edge_index, node2graph,
           WA, bA, WB, bB, WC, bC, WD, bD, WE, bE, WF, bF, WG, bG, WH, bH,
           WI, bI, gamma_h, beta_h, gamma_e, beta_e, gamma_u, beta_u):
    src = edge_index[0]
    dst = edge_index[1]
    n2g_col = node2graph[:, None]
    n2g_row = node2graph[None, :]
    r2 = lambda v: v[None, :]

    t1, t2, eh, xb = _tables_call(
        h, u, n2g_col, WA, r2(bA), WC, r2(bC), WE, r2(bE), WD, r2(bD),
        WF, r2(bF))
    z = _z_call(e, WB, r2(bB))
    r, stats = _sc_gather_call(z, src, dst, t1, t2)
    e_new, sig = _enew_call(r, stats, r2(gamma_e), r2(beta_e))
    num, den, sb, mc = _sc_scatter_call(sig, e_new, src, dst, node2graph, eh)
    h_new, u_new = _final_call(
        num, den, sb, mc, xb, n2g_row, u,
        WG, r2(bG), WH, r2(bH), WI, r2(bI),
        r2(gamma_h), r2(beta_h), r2(gamma_u), r2(beta_u))
    return (h_new, e_new, u_new)


# speedup vs baseline: 3.8469x; 1.6688x over previous
import functools

import jax
import jax.numpy as jnp
from jax import lax
from jax.experimental import pallas as pl
from jax.experimental.pallas import tpu as pltpu
from jax.experimental.pallas import tpu_sc as plsc

N = 10000
E = 320000
B = 128
D = 128

NC = 2
NS = 16
NW = NC * NS
EW = E // NW
ES = E // NS
KG = 80
KE = 80
LANES = 16

F32 = jnp.float32


def _tables_body(h_ref, u_ref, n2g_ref, WA_ref, bA_ref, WC_ref, bC_ref,
                 WE_ref, bE_ref, WD_ref, bD_ref, WF_ref, bF_ref,
                 t1_ref, t2_ref, eh_ref, xb_ref):
    h = h_ref[...]
    u = u_ref[...]
    ah = jnp.dot(h, WA_ref[...], preferred_element_type=F32) + bA_ref[...]
    cu = jnp.dot(u, WC_ref[...], preferred_element_type=F32) + bC_ref[...]
    fu = jnp.dot(u, WF_ref[...], preferred_element_type=F32) + bF_ref[...]
    oh = (n2g_ref[...] == lax.broadcasted_iota(jnp.int32, (N, B), 1)).astype(F32)
    cua = jnp.dot(oh, cu, preferred_element_type=F32)
    h2 = jnp.dot(oh, fu, preferred_element_type=F32)
    dh = jnp.dot(h, WD_ref[...], preferred_element_type=F32) + bD_ref[...]
    t2_ref[...] = ah
    t1_ref[...] = ah + cua
    eh_ref[...] = jnp.dot(h, WE_ref[...], preferred_element_type=F32) + bE_ref[...]
    xb_ref[...] = dh + h2


def _tables_call(h, u, n2g_col, WA, bA, WC, bC, WE, bE, WD, bD, WF, bF):
    out = jax.ShapeDtypeStruct((N, D), F32)
    return pl.pallas_call(
        _tables_body,
        out_shape=(out, out, out, out),
    )(h, u, n2g_col, WA, bA, WC, bC, WE, bE, WD, bD, WF, bF)


_CE = 2000


def _z_body(e_ref, WB_ref, bB_ref, z_ref):
    z_ref[...] = jnp.dot(e_ref[...], WB_ref[...], preferred_element_type=F32) + bB_ref[...]


def _z_call(e, WB, bB):
    return pl.pallas_call(
        _z_body,
        grid=(E // _CE,),
        in_specs=[
            pl.BlockSpec((_CE, D), lambda i: (i, 0)),
            pl.BlockSpec((D, D), lambda i: (0, 0)),
            pl.BlockSpec((1, D), lambda i: (0, 0)),
        ],
        out_specs=pl.BlockSpec((_CE, D), lambda i: (i, 0)),
        out_shape=jax.ShapeDtypeStruct((E, D), F32),
    )(e, WB, bB)


def _sc_gather_body(z_hbm, src_hbm, dst_hbm, t1_hbm, t2_hbm,
                    r_hbm, stats_hbm,
                    src0, src1, dst0, dst1, g10, g11, g20, g21,
                    zb0, zb1, rb0, rb1, acc_v,
                    ssi0, ssi1, sdi0, sdi1, sz0, sz1,
                    sg10, sg11, sg20, sg21, sst0, sst1):
    c = lax.axis_index("c")
    s = lax.axis_index("s")
    wid = s * NC + c
    base_w = wid * EW
    n = EW // KG
    srcs = (src0, src1)
    dsts = (dst0, dst1)
    g1s = (g10, g11)
    g2s = (g20, g21)
    zbs = (zb0, zb1)
    rbs = (rb0, rb1)
    ssi = (ssi0, ssi1)
    sdi = (sdi0, sdi1)
    sz = (sz0, sz1)
    sg1 = (sg10, sg11)
    sg2 = (sg20, sg21)
    sst = (sst0, sst1)

    zero = jnp.zeros((LANES,), F32)
    for cc in range(D // LANES):
        acc_v[0, pl.ds(cc * LANES, LANES)] = zero
        acc_v[1, pl.ds(cc * LANES, LANES)] = zero

    def issue_in(k, b):
        base = base_w + k * KG
        pltpu.async_copy(src_hbm.at[pl.ds(base, KG)], srcs[b], ssi[b])
        pltpu.async_copy(dst_hbm.at[pl.ds(base, KG)], dsts[b], sdi[b])
        pltpu.async_copy(z_hbm.at[pl.ds(base, KG)], zbs[b], sz[b])

    def wait_idx(b):
        pltpu.make_async_copy(src_hbm.at[pl.ds(0, KG)], srcs[b], ssi[b]).wait()
        pltpu.make_async_copy(dst_hbm.at[pl.ds(0, KG)], dsts[b], sdi[b]).wait()

    def issue_gather(b):
        pltpu.async_copy(t1_hbm.at[srcs[b]], g1s[b], sg1[b])
        pltpu.async_copy(t2_hbm.at[dsts[b]], g2s[b], sg2[b])

    def wait_data(b):
        pltpu.make_async_copy(z_hbm.at[pl.ds(0, KG)], zbs[b], sz[b]).wait()
        pltpu.make_async_copy(z_hbm.at[pl.ds(0, KG)], g1s[b], sg1[b]).wait()
        pltpu.make_async_copy(z_hbm.at[pl.ds(0, KG)], g2s[b], sg2[b]).wait()

    def wait_store(b):
        pltpu.make_async_copy(rbs[b], r_hbm.at[pl.ds(0, KG)], sst[b]).wait()

    issue_in(0, 0)
    issue_in(1, 1)
    wait_idx(0)
    issue_gather(0)

    def pair(jp, carry):
        for b in range(2):
            k = 2 * jp + b
            o = 1 - b

            @pl.when(k < n)
            def _():
                @pl.when(k + 1 < n)
                def _():
                    wait_idx(o)
                    issue_gather(o)

                @pl.when(k >= 2)
                def _():
                    wait_store(b)

                wait_data(b)

                def row(i, rc):
                    news = []
                    newq = []
                    for cc in range(D // LANES):
                        sl = pl.ds(cc * LANES, LANES)
                        r = zbs[b][i, sl] + g1s[b][i, sl] + g2s[b][i, sl]
                        rbs[b][i, sl] = r
                        news.append(rc[cc] + r)
                        newq.append(rc[8 + cc] + r * r)
                    return tuple(news) + tuple(newq)

                fin = lax.fori_loop(0, KG, row, (zero,) * 16, unroll=False)
                for cc in range(D // LANES):
                    sl = pl.ds(cc * LANES, LANES)
                    acc_v[0, sl] = acc_v[0, sl] + fin[cc]
                    acc_v[1, sl] = acc_v[1, sl] + fin[8 + cc]

                pltpu.async_copy(
                    rbs[b], r_hbm.at[pl.ds(base_w + k * KG, KG)], sst[b])

                @pl.when(k + 2 < n)
                def _():
                    issue_in(k + 2, b)

        return carry

    lax.fori_loop(0, (n + 1) // 2, pair, 0, unroll=False)
    wait_store(0)
    wait_store(1)
    pltpu.sync_copy(acc_v, stats_hbm.at[wid])


def _sc_gather_call(z, src, dst, t1, t2):
    mesh = plsc.VectorSubcoreMesh(core_axis_name="c", subcore_axis_name="s")
    fn = functools.partial(
        pl.kernel,
        out_type=(
            jax.ShapeDtypeStruct((E, D), F32),
            jax.ShapeDtypeStruct((NW, 2, D), F32),
        ),
        mesh=mesh,
        scratch_types=(
            pltpu.VMEM((KG,), jnp.int32),
            pltpu.VMEM((KG,), jnp.int32),
            pltpu.VMEM((KG,), jnp.int32),
            pltpu.VMEM((KG,), jnp.int32),
            pltpu.VMEM((KG, D), F32),
            pltpu.VMEM((KG, D), F32),
            pltpu.VMEM((KG, D), F32),
            pltpu.VMEM((KG, D), F32),
            pltpu.VMEM((KG, D), F32),
            pltpu.VMEM((KG, D), F32),
            pltpu.VMEM((KG, D), F32),
            pltpu.VMEM((KG, D), F32),
            pltpu.VMEM((2, D), F32),
        ) + (pltpu.SemaphoreType.DMA,) * 12,
    )(_sc_gather_body)
    return fn(z, src, dst, t1, t2)


def _enew_body(r_ref, stats_ref, ge_ref, be_ref, en_ref, sg_ref):
    st = stats_ref[...]
    s0 = jnp.sum(st[:, 0, :], axis=0)
    s1 = jnp.sum(st[:, 1, :], axis=0)
    mu = s0 / float(E)
    var = s1 / float(E) - mu * mu
    rstd = lax.rsqrt(var + 1e-5)
    scale = (ge_ref[...][0] * rstd)[None, :]
    shift = (be_ref[...][0] - mu * ge_ref[...][0] * rstd)[None, :]
    en = jnp.maximum(r_ref[...] * scale + shift, 0.0)
    en_ref[...] = en
    sg_ref[...] = 1.0 / (1.0 + jnp.exp(-en))


def _enew_call(r, stats, gamma_e, beta_e):
    return pl.pallas_call(
        _enew_body,
        grid=(E // _CE,),
        in_specs=[
            pl.BlockSpec((_CE, D), lambda i: (i, 0)),
            pl.BlockSpec((NW, 2, D), lambda i: (0, 0, 0)),
            pl.BlockSpec((1, D), lambda i: (0, 0)),
            pl.BlockSpec((1, D), lambda i: (0, 0)),
        ],
        out_specs=(
            pl.BlockSpec((_CE, D), lambda i: (i, 0)),
            pl.BlockSpec((_CE, D), lambda i: (i, 0)),
        ),
        out_shape=(
            jax.ShapeDtypeStruct((E, D), F32),
            jax.ShapeDtypeStruct((E, D), F32),
        ),
    )(r, stats, gamma_e, beta_e)


_ZC = 80
_NZC = (N // _ZC + NS - 1) // NS


def _sc_scatter_body(sig_hbm, en_hbm, src_hbm, dst_hbm, n2g_hbm, eh_hbm,
                     num_hbm, den_hbm, sb_hbm, mc_hbm,
                     tbl, sbt, mct,
                     src_v, dst_v, g2_v, a_v, b_v, ones_v, z16_v,
                     sem1, sem2, sem3):
    c = lax.axis_index("c")
    s = lax.axis_index("s")
    sid = s
    zero = jnp.zeros((LANES,), F32)
    one = jnp.full((LANES,), 1.0, F32)

    def zrow(i, carry):
        for cc in range(D // LANES):
            a_v[i, pl.ds(cc * LANES, LANES)] = zero
        return carry

    lax.fori_loop(0, KE, zrow, 0, unroll=False)

    def orow(i, carry):
        ones_v[i, :] = one
        return carry

    lax.fori_loop(0, KE, orow, 0, unroll=False)

    def z16row(i, carry):
        z16_v[i, :] = zero
        return carry

    lax.fori_loop(0, B, z16row, 0, unroll=False)

    nzc = N // _ZC

    def zchunk(k, carry):
        j = sid + k * NS

        @pl.when(j < nzc)
        def _():
            pltpu.sync_copy(a_v.at[pl.ds(0, _ZC)], tbl.at[pl.ds(j * _ZC, _ZC)])

        return carry

    lax.fori_loop(0, _NZC, zchunk, 0, unroll=False)

    @pl.when(sid == 0)
    def _():
        pltpu.sync_copy(a_v.at[pl.ds(0, B)], sbt)
        pltpu.sync_copy(z16_v, mct)

    plsc.subcore_barrier()

    nchunk = ES // KE

    @pl.when(c == 0)
    def _():
        def chunk(j, carry):
            base = s * ES + j * KE
            ci1 = pltpu.async_copy(src_hbm.at[pl.ds(base, KE)], src_v, sem1)
            ci2 = pltpu.async_copy(dst_hbm.at[pl.ds(base, KE)], dst_v, sem2)
            cpa = pltpu.async_copy(sig_hbm.at[pl.ds(base, KE)], a_v, sem3)
            ci1.wait()
            cp = pltpu.async_copy(eh_hbm.at[src_v], b_v, sem1)
            ci2.wait()
            cpa.wait()
            cp.wait()

            def row(i, rc):
                for cc in range(D // LANES):
                    sl = pl.ds(cc * LANES, LANES)
                    a_v[i, sl] = a_v[i, sl] * b_v[i, sl]
                return rc

            lax.fori_loop(0, KE, row, 0, unroll=False)
            pltpu.sync_copy(a_v, tbl.at[dst_v], add=True)
            return carry

        lax.fori_loop(0, nchunk, chunk, 0, unroll=False)

    @pl.when(c == 1)
    def _():
        def chunk(j, carry):
            base = s * ES + j * KE
            ci2 = pltpu.async_copy(dst_hbm.at[pl.ds(base, KE)], dst_v, sem2)
            cpa = pltpu.async_copy(sig_hbm.at[pl.ds(base, KE)], a_v, sem3)
            cpb = pltpu.async_copy(en_hbm.at[pl.ds(base, KE)], b_v, sem1)
            ci2.wait()
            cpg = pltpu.async_copy(n2g_hbm.at[dst_v], g2_v, sem2)
            cpa.wait()
            pltpu.sync_copy(a_v, tbl.at[dst_v], add=True)
            cpg.wait()
            cpb.wait()
            pltpu.sync_copy(b_v, sbt.at[g2_v], add=True)
            pltpu.sync_copy(ones_v, mct.at[g2_v], add=True)
            return carry

        lax.fori_loop(0, nchunk, chunk, 0, unroll=False)

    plsc.subcore_barrier()

    @pl.when(sid == 0)
    def _():
        @pl.when(c == 0)
        def _():
            pltpu.sync_copy(tbl, num_hbm)

        @pl.when(c == 1)
        def _():
            pltpu.sync_copy(tbl, den_hbm)
            pltpu.sync_copy(sbt, sb_hbm)
            pltpu.sync_copy(mct, mc_hbm)


def _sc_scatter_call(sig, en, src, dst, n2g, eh):
    mesh = plsc.VectorSubcoreMesh(core_axis_name="c", subcore_axis_name="s")
    fn = functools.partial(
        pl.kernel,
        out_type=(
            jax.ShapeDtypeStruct((N, D), F32),
            jax.ShapeDtypeStruct((N, D), F32),
            jax.ShapeDtypeStruct((B, D), F32),
            jax.ShapeDtypeStruct((B, LANES), F32),
        ),
        mesh=mesh,
        scratch_types=(
            pltpu.VMEM_SHARED((N, D), F32),
            pltpu.VMEM_SHARED((B, D), F32),
            pltpu.VMEM_SHARED((B, LANES), F32),
            pltpu.VMEM((KE,), jnp.int32),
            pltpu.VMEM((KE,), jnp.int32),
            pltpu.VMEM((KE,), jnp.int32),
            pltpu.VMEM((KE, D), F32),
            pltpu.VMEM((KE, D), F32),
            pltpu.VMEM((KE, LANES), F32),
            pltpu.VMEM((B, LANES), F32),
            pltpu.SemaphoreType.DMA,
            pltpu.SemaphoreType.DMA,
            pltpu.SemaphoreType.DMA,
        ),
    )(_sc_scatter_body)
    return fn(sig, en, src, dst, n2g, eh)


def _final_body(num_ref, den_ref, sb_ref, mc_ref, xb_ref, n2g_ref, u_ref,
                WG_ref, bG_ref, WH_ref, bH_ref, WI_ref, bI_ref,
                gh_ref, bh_ref, gu_ref, bu_ref,
                hn_ref, un_ref):
    x = xb_ref[...] + num_ref[...] / (den_ref[...] + 1e-6)
    mu = jnp.mean(x, axis=0, keepdims=True)
    var = jnp.mean(x * x, axis=0, keepdims=True) - mu * mu
    hn = gh_ref[...] * (x - mu) * lax.rsqrt(var + 1e-5) + bh_ref[...]
    hn = jnp.maximum(hn, 0.0)
    hn_ref[...] = hn

    ohT = (lax.broadcasted_iota(jnp.int32, (B, N), 0) == n2g_ref[...]).astype(F32)
    M = jnp.dot(ohT, hn, preferred_element_type=F32)
    cnt = jnp.sum(ohT, axis=1, keepdims=True)
    Gg = jnp.dot(M, WG_ref[...], preferred_element_type=F32) + cnt * bG_ref[...]
    mean_Gh = Gg / jnp.maximum(cnt, 1.0)
    mg = jnp.sum(mc_ref[...], axis=1, keepdims=True) * (1.0 / LANES)
    He_g = jnp.dot(sb_ref[...], WH_ref[...], preferred_element_type=F32) + mg * bH_ref[...]
    mean_He = He_g * (1.0 / float(E))
    un = mean_Gh + mean_He + jnp.dot(u_ref[...], WI_ref[...], preferred_element_type=F32) + bI_ref[...]
    mu_u = jnp.mean(un, axis=0, keepdims=True)
    var_u = jnp.mean(un * un, axis=0, keepdims=True) - mu_u * mu_u
    un = gu_ref[...] * (un - mu_u) * lax.rsqrt(var_u + 1e-5) + bu_ref[...]
    un_ref[...] = jnp.maximum(un, 0.0)


def _final_call(num, den, sb, mc, xb, n2g_row, u,
                WG, bG, WH, bH, WI, bI, gamma_h, beta_h, gamma_u, beta_u):
    return pl.pallas_call(
        _final_body,
        out_shape=(
            jax.ShapeDtypeStruct((N, D), F32),
            jax.ShapeDtypeStruct((B, D), F32),
        ),
    )(num, den, sb, mc, xb, n2g_row, u,
      WG, bG, WH, bH, WI, bI, gamma_h, beta_h, gamma_u, beta_u)


def kernel(h, e, u, edge_index, node2graph,
           WA, bA, WB, bB, WC, bC, WD, bD, WE, bE, WF, bF, WG, bG, WH, bH,
           WI, bI, gamma_h, beta_h, gamma_e, beta_e, gamma_u, beta_u):
    src = edge_index[0]
    dst = edge_index[1]
    n2g_col = node2graph[:, None]
    n2g_row = node2graph[None, :]
    r2 = lambda v: v[None, :]

    t1, t2, eh, xb = _tables_call(
        h, u, n2g_col, WA, r2(bA), WC, r2(bC), WE, r2(bE), WD, r2(bD),
        WF, r2(bF))
    z = _z_call(e, WB, r2(bB))
    r, stats = _sc_gather_call(z, src, dst, t1, t2)
    e_new, sig = _enew_call(r, stats, r2(gamma_e), r2(beta_e))
    num, den, sb, mc = _sc_scatter_call(sig, e_new, src, dst, node2graph, eh)
    h_new, u_new = _final_call(
        num, den, sb, mc, xb, n2g_row, u,
        WG, r2(bG), WH, r2(bH), WI, r2(bI),
        r2(gamma_h), r2(beta_h), r2(gamma_u), r2(beta_u))
    return (h_new, e_new, u_new)


# speedup vs baseline: 4.3023x; 1.1184x over previous
import functools

import jax
import jax.numpy as jnp
from jax import lax
from jax.experimental import pallas as pl
from jax.experimental.pallas import tpu as pltpu
from jax.experimental.pallas import tpu_sc as plsc

N = 10000
E = 320000
B = 128
D = 128

NC = 2
NS = 16
NW = NC * NS
EW = E // NW
ES = E // NS
KG = 80
KE = 80
LANES = 16

F32 = jnp.float32


def _tables_body(h_ref, u_ref, n2g_ref, WA_ref, bA_ref, WC_ref, bC_ref,
                 WE_ref, bE_ref, WD_ref, bD_ref, WF_ref, bF_ref,
                 t1_ref, t2_ref, eh_ref, xb_ref):
    h = h_ref[...]
    u = u_ref[...]
    ah = jnp.dot(h, WA_ref[...], preferred_element_type=F32) + bA_ref[...]
    cu = jnp.dot(u, WC_ref[...], preferred_element_type=F32) + bC_ref[...]
    fu = jnp.dot(u, WF_ref[...], preferred_element_type=F32) + bF_ref[...]
    oh = (n2g_ref[...] == lax.broadcasted_iota(jnp.int32, (N, B), 1)).astype(F32)
    cua = jnp.dot(oh, cu, preferred_element_type=F32)
    h2 = jnp.dot(oh, fu, preferred_element_type=F32)
    dh = jnp.dot(h, WD_ref[...], preferred_element_type=F32) + bD_ref[...]
    t2_ref[...] = ah
    t1_ref[...] = ah + cua
    eh_ref[...] = jnp.dot(h, WE_ref[...], preferred_element_type=F32) + bE_ref[...]
    xb_ref[...] = dh + h2


def _tables_call(h, u, n2g_col, WA, bA, WC, bC, WE, bE, WD, bD, WF, bF):
    out = jax.ShapeDtypeStruct((N, D), F32)
    return pl.pallas_call(
        _tables_body,
        out_shape=(out, out, out, out),
    )(h, u, n2g_col, WA, bA, WC, bC, WE, bE, WD, bD, WF, bF)


_CE = 2000


def _z_body(e_ref, WB_ref, bB_ref, z_ref):
    z_ref[...] = jnp.dot(e_ref[...], WB_ref[...], preferred_element_type=F32) + bB_ref[...]


def _z_call(e, WB, bB):
    return pl.pallas_call(
        _z_body,
        grid=(E // _CE,),
        in_specs=[
            pl.BlockSpec((_CE, D), lambda i: (i, 0)),
            pl.BlockSpec((D, D), lambda i: (0, 0)),
            pl.BlockSpec((1, D), lambda i: (0, 0)),
        ],
        out_specs=pl.BlockSpec((_CE, D), lambda i: (i, 0)),
        out_shape=jax.ShapeDtypeStruct((E, D), F32),
    )(e, WB, bB)


def _sc_gather_body(z_hbm, src_hbm, dst_hbm, t1_hbm, t2_hbm,
                    r_hbm, stats_hbm,
                    src0, src1, dst0, dst1, g10, g11, g20, g21,
                    zb0, zb1, rb0, rb1, acc_v,
                    ssi0, ssi1, sdi0, sdi1, sz0, sz1,
                    sg10, sg11, sg20, sg21, sst0, sst1):
    c = lax.axis_index("c")
    s = lax.axis_index("s")
    wid = s * NC + c
    base_w = wid * EW
    n = EW // KG
    srcs = (src0, src1)
    dsts = (dst0, dst1)
    g1s = (g10, g11)
    g2s = (g20, g21)
    zbs = (zb0, zb1)
    rbs = (rb0, rb1)
    ssi = (ssi0, ssi1)
    sdi = (sdi0, sdi1)
    sz = (sz0, sz1)
    sg1 = (sg10, sg11)
    sg2 = (sg20, sg21)
    sst = (sst0, sst1)

    zero = jnp.zeros((LANES,), F32)
    for cc in range(D // LANES):
        acc_v[0, pl.ds(cc * LANES, LANES)] = zero
        acc_v[1, pl.ds(cc * LANES, LANES)] = zero

    def issue_in(k, b):
        base = base_w + k * KG
        pltpu.async_copy(src_hbm.at[pl.ds(base, KG)], srcs[b], ssi[b])
        pltpu.async_copy(dst_hbm.at[pl.ds(base, KG)], dsts[b], sdi[b])
        pltpu.async_copy(z_hbm.at[pl.ds(base, KG)], zbs[b], sz[b])

    def wait_idx(b):
        pltpu.make_async_copy(src_hbm.at[pl.ds(0, KG)], srcs[b], ssi[b]).wait()
        pltpu.make_async_copy(dst_hbm.at[pl.ds(0, KG)], dsts[b], sdi[b]).wait()

    def issue_gather(b):
        pltpu.async_copy(t1_hbm.at[srcs[b]], g1s[b], sg1[b])
        pltpu.async_copy(t2_hbm.at[dsts[b]], g2s[b], sg2[b])

    def wait_data(b):
        pltpu.make_async_copy(z_hbm.at[pl.ds(0, KG)], zbs[b], sz[b]).wait()
        pltpu.make_async_copy(z_hbm.at[pl.ds(0, KG)], g1s[b], sg1[b]).wait()
        pltpu.make_async_copy(z_hbm.at[pl.ds(0, KG)], g2s[b], sg2[b]).wait()

    def wait_store(b):
        pltpu.make_async_copy(rbs[b], r_hbm.at[pl.ds(0, KG)], sst[b]).wait()

    issue_in(0, 0)
    issue_in(1, 1)
    wait_idx(0)
    issue_gather(0)

    def pair(jp, carry):
        for b in range(2):
            k = 2 * jp + b
            o = 1 - b

            @pl.when(k < n)
            def _():
                @pl.when(k + 1 < n)
                def _():
                    wait_idx(o)
                    issue_gather(o)

                @pl.when(k >= 2)
                def _():
                    wait_store(b)

                wait_data(b)

                def row(i, rc):
                    news = []
                    newq = []
                    for cc in range(D // LANES):
                        sl = pl.ds(cc * LANES, LANES)
                        r = zbs[b][i, sl] + g1s[b][i, sl] + g2s[b][i, sl]
                        rbs[b][i, sl] = r
                        news.append(rc[cc] + r)
                        newq.append(rc[8 + cc] + r * r)
                    return tuple(news) + tuple(newq)

                fin = lax.fori_loop(0, KG, row, (zero,) * 16, unroll=False)
                for cc in range(D // LANES):
                    sl = pl.ds(cc * LANES, LANES)
                    acc_v[0, sl] = acc_v[0, sl] + fin[cc]
                    acc_v[1, sl] = acc_v[1, sl] + fin[8 + cc]

                pltpu.async_copy(
                    rbs[b], r_hbm.at[pl.ds(base_w + k * KG, KG)], sst[b])

                @pl.when(k + 2 < n)
                def _():
                    issue_in(k + 2, b)

        return carry

    lax.fori_loop(0, (n + 1) // 2, pair, 0, unroll=False)
    wait_store(0)
    wait_store(1)
    pltpu.sync_copy(acc_v, stats_hbm.at[wid])


def _sc_gather_call(z, src, dst, t1, t2):
    mesh = plsc.VectorSubcoreMesh(core_axis_name="c", subcore_axis_name="s")
    fn = functools.partial(
        pl.kernel,
        out_type=(
            jax.ShapeDtypeStruct((E, D), F32),
            jax.ShapeDtypeStruct((NW, 2, D), F32),
        ),
        mesh=mesh,
        scratch_types=(
            pltpu.VMEM((KG,), jnp.int32),
            pltpu.VMEM((KG,), jnp.int32),
            pltpu.VMEM((KG,), jnp.int32),
            pltpu.VMEM((KG,), jnp.int32),
            pltpu.VMEM((KG, D), F32),
            pltpu.VMEM((KG, D), F32),
            pltpu.VMEM((KG, D), F32),
            pltpu.VMEM((KG, D), F32),
            pltpu.VMEM((KG, D), F32),
            pltpu.VMEM((KG, D), F32),
            pltpu.VMEM((KG, D), F32),
            pltpu.VMEM((KG, D), F32),
            pltpu.VMEM((2, D), F32),
        ) + (pltpu.SemaphoreType.DMA,) * 12,
    )(_sc_gather_body)
    return fn(z, src, dst, t1, t2)


def _enew_body(r_ref, stats_ref, ge_ref, be_ref,
               en_ref, sgl_ref, sgr_ref):
    st = stats_ref[...]
    s0 = jnp.sum(st[:, 0, :], axis=0)
    s1 = jnp.sum(st[:, 1, :], axis=0)
    mu = s0 / float(E)
    var = s1 / float(E) - mu * mu
    rstd = lax.rsqrt(var + 1e-5)
    scale = (ge_ref[...][0] * rstd)[None, :]
    shift = (be_ref[...][0] - mu * ge_ref[...][0] * rstd)[None, :]
    en = jnp.maximum(r_ref[...] * scale + shift, 0.0)
    sg = 1.0 / (1.0 + jnp.exp(-en))
    en_ref[...] = en
    sgl_ref[...] = sg[:, :H]
    sgr_ref[...] = sg[:, H:]


def _enew_call(r, stats, gamma_e, beta_e):
    half = pl.BlockSpec((_CE, D // 2), lambda i: (i, 0))
    return pl.pallas_call(
        _enew_body,
        grid=(E // _CE,),
        in_specs=[
            pl.BlockSpec((_CE, D), lambda i: (i, 0)),
            pl.BlockSpec((NW, 2, D), lambda i: (0, 0, 0)),
            pl.BlockSpec((1, D), lambda i: (0, 0)),
            pl.BlockSpec((1, D), lambda i: (0, 0)),
        ],
        out_specs=(
            pl.BlockSpec((_CE, D), lambda i: (i, 0)),
            half, half,
        ),
        out_shape=(
            jax.ShapeDtypeStruct((E, D), F32),
            jax.ShapeDtypeStruct((E, H), F32),
            jax.ShapeDtypeStruct((E, H), F32),
        ),
    )(r, stats, gamma_e, beta_e)


H = D // 2


def _sc_scatter_body(sgl_hbm, sgr_hbm, src_hbm, dst_hbm, eh_hbm,
                     nd0_hbm, nd1_hbm,
                     ndT,
                     src0, src1, dst0, dst1,
                     a0, a1, g0, g1,
                     ssi0, ssi1, sdi0, sdi1, sa0, sa1,
                     sg0, sg1, scn0, scn1):
    c = lax.axis_index("c")
    s = lax.axis_index("s")
    srcs = (src0, src1)
    dsts = (dst0, dst1)
    aas = (a0, a1)
    gs = (g0, g1)
    ssi = (ssi0, ssi1)
    sdi = (sdi0, sdi1)
    sa = (sa0, sa1)
    sg = (sg0, sg1)
    scn = (scn0, scn1)
    n = ES // KE
    zero = jnp.zeros((LANES,), F32)

    def zrow(i, carry):
        for cc in range(D // LANES):
            g0[i, pl.ds(cc * LANES, LANES)] = zero
        return carry

    lax.fori_loop(0, KE, zrow, 0, unroll=False)

    nzc = N // KE

    def zchunk(k, carry):
        j = s + k * NS

        @pl.when(j < nzc)
        def _():
            pltpu.sync_copy(g0, ndT.at[pl.ds(j * KE, KE)])

        return carry

    lax.fori_loop(0, (nzc + NS - 1) // NS, zchunk, 0, unroll=False)
    plsc.subcore_barrier()

    def run_edges(sg_t, off):
        base_s = s * ES

        def issue_in(k, b):
            base = base_s + k * KE
            pltpu.async_copy(src_hbm.at[pl.ds(base, KE)], srcs[b], ssi[b])
            pltpu.async_copy(dst_hbm.at[pl.ds(base, KE)], dsts[b], sdi[b])
            pltpu.async_copy(sg_t.at[pl.ds(base, KE)], aas[b], sa[b])

        def wait_idx(b):
            pltpu.make_async_copy(src_hbm.at[pl.ds(0, KE)], srcs[b], ssi[b]).wait()
            pltpu.make_async_copy(dst_hbm.at[pl.ds(0, KE)], dsts[b], sdi[b]).wait()

        def issue_gather(b):
            pltpu.async_copy(eh_hbm.at[srcs[b]], gs[b], sg[b])

        def wait_scn(b):
            pltpu.make_async_copy(gs[b], ndT.at[pl.ds(0, KE)], scn[b]).wait()

        issue_in(0, 0)
        issue_in(1, 1)
        wait_idx(0)
        issue_gather(0)

        def pair(jp, carry):
            for b in range(2):
                k = 2 * jp + b
                o = 1 - b

                @pl.when(k < n)
                def _():
                    @pl.when(k + 1 < n)
                    def _():
                        wait_idx(o)
                        issue_gather(o)

                    pltpu.make_async_copy(
                        sg_t.at[pl.ds(0, KE)], aas[b], sa[b]).wait()
                    pltpu.make_async_copy(
                        eh_hbm.at[pl.ds(0, KE)], gs[b], sg[b]).wait()

                    def row(i, rc):
                        for cc in range(H // LANES):
                            sl = pl.ds(cc * LANES, LANES)
                            slo = pl.ds(off + cc * LANES, LANES)
                            slr = pl.ds(H + cc * LANES, LANES)
                            sv = aas[b][i, sl]
                            p = gs[b][i, slo] * sv
                            gs[b][i, sl] = p
                            gs[b][i, slr] = sv
                        return rc

                    lax.fori_loop(0, KE, row, 0, unroll=False)
                    pltpu.async_copy(gs[b], ndT.at[dsts[b]], scn[b], add=True)

                    @pl.when(k + 2 < n)
                    def _():
                        wait_scn(b)
                        issue_in(k + 2, b)

            return carry

        lax.fori_loop(0, (n + 1) // 2, pair, 0, unroll=False)
        wait_scn(0)
        wait_scn(1)

    @pl.when(c == 0)
    def _():
        run_edges(sgl_hbm, 0)

    @pl.when(c == 1)
    def _():
        run_edges(sgr_hbm, H)

    plsc.subcore_barrier()

    @pl.when(s == 0)
    def _():
        @pl.when(c == 0)
        def _():
            pltpu.sync_copy(ndT, nd0_hbm)

        @pl.when(c == 1)
        def _():
            pltpu.sync_copy(ndT, nd1_hbm)


def _sc_scatter_call(sgl, sgr, src, dst, eh):
    mesh = plsc.VectorSubcoreMesh(core_axis_name="c", subcore_axis_name="s")
    outn = jax.ShapeDtypeStruct((N, D), F32)
    fn = functools.partial(
        pl.kernel,
        out_type=(outn, outn),
        mesh=mesh,
        scratch_types=(
            pltpu.VMEM_SHARED((N, D), F32),
            pltpu.VMEM((KE,), jnp.int32),
            pltpu.VMEM((KE,), jnp.int32),
            pltpu.VMEM((KE,), jnp.int32),
            pltpu.VMEM((KE,), jnp.int32),
            pltpu.VMEM((KE, H), F32),
            pltpu.VMEM((KE, H), F32),
            pltpu.VMEM((KE, D), F32),
            pltpu.VMEM((KE, D), F32),
        ) + (pltpu.SemaphoreType.DMA,) * 10,
    )(_sc_scatter_body)
    return fn(sgl, sgr, src, dst, eh)


ES2 = E // (2 * NS)


def _sc_sb_body(en_hbm, dst_hbm, n2g_hbm,
                sb0_hbm, sb1_hbm, mc0_hbm, mc1_hbm,
                sbt, mct,
                dst0, dst1, gg0, gg1, e0, e1, ones_v,
                sdi0, sdi1, sgg0, sgg1, se0, se1,
                scs0, scs1, scm0, scm1):
    c = lax.axis_index("c")
    s = lax.axis_index("s")
    dsts = (dst0, dst1)
    ggs = (gg0, gg1)
    ees = (e0, e1)
    sdi = (sdi0, sdi1)
    sgg = (sgg0, sgg1)
    se = (se0, se1)
    scs = (scs0, scs1)
    scm = (scm0, scm1)
    n = ES2 // KE
    zero = jnp.zeros((LANES,), F32)
    one = jnp.full((LANES,), 1.0, F32)

    def zrow(i, carry):
        for cc in range(D // LANES):
            e0[i, pl.ds(cc * LANES, LANES)] = zero
            ones_v[i, pl.ds(cc * LANES, LANES)] = zero
        return carry

    lax.fori_loop(0, KE, zrow, 0, unroll=False)

    @pl.when(s == 0)
    def _():
        pltpu.sync_copy(e0, sbt.at[pl.ds(0, KE)])
        pltpu.sync_copy(e0.at[pl.ds(0, B - KE)], sbt.at[pl.ds(KE, B - KE)])
        pltpu.sync_copy(ones_v, mct.at[pl.ds(0, KE)])
        pltpu.sync_copy(ones_v.at[pl.ds(0, B - KE)], mct.at[pl.ds(KE, B - KE)])

    def orow(i, carry):
        for cc in range(D // LANES):
            ones_v[i, pl.ds(cc * LANES, LANES)] = one
        return carry

    lax.fori_loop(0, KE, orow, 0, unroll=False)
    plsc.subcore_barrier()

    base_s = c * (E // 2) + s * ES2

    def issue_in(k, b):
        base = base_s + k * KE
        pltpu.async_copy(dst_hbm.at[pl.ds(base, KE)], dsts[b], sdi[b])
        pltpu.async_copy(en_hbm.at[pl.ds(base, KE)], ees[b], se[b])

    def wait_idx(b):
        pltpu.make_async_copy(dst_hbm.at[pl.ds(0, KE)], dsts[b], sdi[b]).wait()

    def issue_gather(b):
        pltpu.async_copy(n2g_hbm.at[dsts[b]], ggs[b], sgg[b])

    def wait_scatters(b):
        pltpu.make_async_copy(ees[b], sbt.at[pl.ds(0, KE)], scs[b]).wait()
        pltpu.make_async_copy(ones_v, mct.at[pl.ds(0, KE)], scm[b]).wait()

    issue_in(0, 0)
    issue_in(1, 1)
    wait_idx(0)
    issue_gather(0)

    def pair(jp, carry):
        for b in range(2):
            k = 2 * jp + b
            o = 1 - b

            @pl.when(k < n)
            def _():
                @pl.when(k + 1 < n)
                def _():
                    wait_idx(o)
                    issue_gather(o)

                pltpu.make_async_copy(
                    en_hbm.at[pl.ds(0, KE)], ees[b], se[b]).wait()
                pltpu.make_async_copy(
                    dst_hbm.at[pl.ds(0, KE)], ggs[b], sgg[b]).wait()
                pltpu.async_copy(ees[b], sbt.at[ggs[b]], scs[b], add=True)
                pltpu.async_copy(ones_v, mct.at[ggs[b]], scm[b], add=True)

                @pl.when(k + 2 < n)
                def _():
                    wait_scatters(b)
                    issue_in(k + 2, b)

        return carry

    lax.fori_loop(0, (n + 1) // 2, pair, 0, unroll=False)
    wait_scatters(0)
    wait_scatters(1)
    plsc.subcore_barrier()

    @pl.when(s == 0)
    def _():
        @pl.when(c == 0)
        def _():
            pltpu.sync_copy(sbt, sb0_hbm)
            pltpu.sync_copy(mct, mc0_hbm)

        @pl.when(c == 1)
        def _():
            pltpu.sync_copy(sbt, sb1_hbm)
            pltpu.sync_copy(mct, mc1_hbm)


def _sc_sb_call(en, dst, n2g):
    mesh = plsc.VectorSubcoreMesh(core_axis_name="c", subcore_axis_name="s")
    outb = jax.ShapeDtypeStruct((B, D), F32)
    outm = jax.ShapeDtypeStruct((B, D), F32)
    fn = functools.partial(
        pl.kernel,
        out_type=(outb, outb, outm, outm),
        mesh=mesh,
        scratch_types=(
            pltpu.VMEM_SHARED((B, D), F32),
            pltpu.VMEM_SHARED((B, D), F32),
            pltpu.VMEM((KE,), jnp.int32),
            pltpu.VMEM((KE,), jnp.int32),
            pltpu.VMEM((KE,), jnp.int32),
            pltpu.VMEM((KE,), jnp.int32),
            pltpu.VMEM((KE, D), F32),
            pltpu.VMEM((KE, D), F32),
            pltpu.VMEM((KE, D), F32),
        ) + (pltpu.SemaphoreType.DMA,) * 10,
    )(_sc_sb_body)
    return fn(en, dst, n2g)


def _final_body(nd0_ref, nd1_ref, sb0_ref, sb1_ref,
                mc0_ref, mc1_ref, xb_ref, n2g_ref, u_ref,
                WG_ref, bG_ref, WH_ref, bH_ref, WI_ref, bI_ref,
                gh_ref, bh_ref, gu_ref, bu_ref,
                hn_ref, un_ref):
    nd0 = nd0_ref[...]
    nd1 = nd1_ref[...]
    num = jnp.concatenate([nd0[:, :H], nd1[:, :H]], axis=1)
    den = jnp.concatenate([nd0[:, H:], nd1[:, H:]], axis=1)
    sb_full = sb0_ref[...] + sb1_ref[...]
    x = xb_ref[...] + num / (den + 1e-6)
    mu = jnp.mean(x, axis=0, keepdims=True)
    var = jnp.mean(x * x, axis=0, keepdims=True) - mu * mu
    hn = gh_ref[...] * (x - mu) * lax.rsqrt(var + 1e-5) + bh_ref[...]
    hn = jnp.maximum(hn, 0.0)
    hn_ref[...] = hn

    ohT = (lax.broadcasted_iota(jnp.int32, (B, N), 0) == n2g_ref[...]).astype(F32)
    M = jnp.dot(ohT, hn, preferred_element_type=F32)
    cnt = jnp.sum(ohT, axis=1, keepdims=True)
    Gg = jnp.dot(M, WG_ref[...], preferred_element_type=F32) + cnt * bG_ref[...]
    mean_Gh = Gg / jnp.maximum(cnt, 1.0)
    mg = jnp.sum(mc0_ref[...] + mc1_ref[...], axis=1, keepdims=True) * (1.0 / D)
    He_g = jnp.dot(sb_full, WH_ref[...], preferred_element_type=F32) + mg * bH_ref[...]
    mean_He = He_g * (1.0 / float(E))
    un = mean_Gh + mean_He + jnp.dot(u_ref[...], WI_ref[...], preferred_element_type=F32) + bI_ref[...]
    mu_u = jnp.mean(un, axis=0, keepdims=True)
    var_u = jnp.mean(un * un, axis=0, keepdims=True) - mu_u * mu_u
    un = gu_ref[...] * (un - mu_u) * lax.rsqrt(var_u + 1e-5) + bu_ref[...]
    un_ref[...] = jnp.maximum(un, 0.0)


def _final_call(nd0, nd1, sb0, sb1, mc0, mc1, xb, n2g_row, u,
                WG, bG, WH, bH, WI, bI, gamma_h, beta_h, gamma_u, beta_u):
    return pl.pallas_call(
        _final_body,
        out_shape=(
            jax.ShapeDtypeStruct((N, D), F32),
            jax.ShapeDtypeStruct((B, D), F32),
        ),
    )(nd0, nd1, sb0, sb1, mc0, mc1, xb, n2g_row, u,
      WG, bG, WH, bH, WI, bI, gamma_h, beta_h, gamma_u, beta_u)


def kernel(h, e, u, edge_index, node2graph,
           WA, bA, WB, bB, WC, bC, WD, bD, WE, bE, WF, bF, WG, bG, WH, bH,
           WI, bI, gamma_h, beta_h, gamma_e, beta_e, gamma_u, beta_u):
    src = edge_index[0]
    dst = edge_index[1]
    n2g_col = node2graph[:, None]
    n2g_row = node2graph[None, :]
    r2 = lambda v: v[None, :]

    t1, t2, eh, xb = _tables_call(
        h, u, n2g_col, WA, r2(bA), WC, r2(bC), WE, r2(bE), WD, r2(bD),
        WF, r2(bF))
    z = _z_call(e, WB, r2(bB))
    r, stats = _sc_gather_call(z, src, dst, t1, t2)
    e_new, sgl, sgr = _enew_call(r, stats, r2(gamma_e), r2(beta_e))
    nd0, nd1 = _sc_scatter_call(sgl, sgr, src, dst, eh)
    sb0, sb1, mc0, mc1 = _sc_sb_call(e_new, dst, node2graph)
    h_new, u_new = _final_call(
        nd0, nd1, sb0, sb1, mc0, mc1, xb, n2g_row, u,
        WG, r2(bG), WH, r2(bH), WI, r2(bI),
        r2(gamma_h), r2(beta_h), r2(gamma_u), r2(beta_u))
    return (h_new, e_new, u_new)
